# Initial kernel scaffold; baseline (speedup 1.0000x reference)
import functools

import jax
import jax.numpy as jnp
from jax import lax
from jax.experimental import pallas as pl
from jax.experimental.pallas import tpu as pltpu
from jax.experimental.pallas import tpu_sc as plsc

NC = 2
NS = 16
NW = NC * NS


def _make_segsum(N, E, D):
  ep = E // NW
  ch = 80
  nchunk = ep // ch
  rows_per_tile = N // NS
  zch = 125
  nz = rows_per_tile // zch

  mesh = plsc.VectorSubcoreMesh(core_axis_name="c", subcore_axis_name="s")

  @functools.partial(
      pl.kernel,
      out_type=jax.ShapeDtypeStruct((NC, N, D), jnp.float32),
      mesh=mesh,
      scratch_types=[
          pltpu.VMEM((ch,), jnp.int32),
          pltpu.VMEM((ch,), jnp.int32),
          pltpu.VMEM((ch, D), jnp.float32),
          pltpu.VMEM((zch, D), jnp.float32),
          pltpu.VMEM_SHARED((N, D), jnp.float32),
          pltpu.SemaphoreType.DMA,
      ],
  )
  def segsum(src_hbm, dst_hbm, x_hbm, out_hbm, sidx, didx, rows, zbuf, yacc,
             sem):
    c = lax.axis_index("c")
    s = lax.axis_index("s")
    wid = c * NS + s

    zv = jnp.zeros((16,), jnp.float32)

    def zrow(r, carry):
      for k in range(D // 16):
        zbuf[r, pl.ds(k * 16, 16)] = zv
      return carry

    lax.fori_loop(0, zch, zrow, 0)

    row0 = s * rows_per_tile
    for z in range(nz):
      pltpu.sync_copy(zbuf, yacc.at[pl.ds(row0 + z * zch, zch)])

    plsc.subcore_barrier()

    ebase = wid * ep

    def edge_chunk(j, carry):
      off = pl.multiple_of(ebase + j * ch, 8)
      pltpu.sync_copy(src_hbm.at[pl.ds(off, ch)], sidx)
      pltpu.sync_copy(dst_hbm.at[pl.ds(off, ch)], didx)
      pltpu.async_copy(x_hbm.at[sidx], rows, sem).wait()
      pltpu.sync_copy(rows, yacc.at[didx], add=True)
      return carry

    lax.fori_loop(0, nchunk, edge_chunk, 0)

    plsc.subcore_barrier()

    for z in range(nz):
      r0 = row0 + z * zch
      pltpu.sync_copy(yacc.at[pl.ds(r0, zch)], out_hbm.at[c, pl.ds(r0, zch)])

  return segsum


def _dense_body(yp_ref, x_ref, w1_ref, b1_ref, w2_ref, b2_ref, eps_ref,
                gamma_ref, beta_ref, o_ref):
  n = x_ref.shape[0]
  y = yp_ref[0] + yp_ref[1]
  h = y + (1.0 + eps_ref[0]) * x_ref[...]
  h = lax.dot_general(h, w1_ref[...], (((1,), (1,)), ((), ())),
                      preferred_element_type=jnp.float32)
  h = jnp.maximum(h + b1_ref[...][None, :], 0.0)
  h = lax.dot_general(h, w2_ref[...], (((1,), (1,)), ((), ())),
                      preferred_element_type=jnp.float32)
  h = h + b2_ref[...][None, :]
  mean = jnp.sum(h, axis=0, keepdims=True) * (1.0 / n)
  d = h - mean
  var = jnp.sum(d * d, axis=0, keepdims=True) * (1.0 / n)
  o_ref[...] = d * lax.rsqrt(var + 1e-5) * gamma_ref[...][None, :] \
      + beta_ref[...][None, :]


def kernel(x, edge_index, W1, b1, W2, b2, eps, gamma, beta):
  N, D = x.shape
  E = edge_index.shape[1]
  src = edge_index[0]
  dst = edge_index[1]

  yp = _make_segsum(N, E, D)(src, dst, x)

  vmem = pl.BlockSpec(memory_space=pltpu.VMEM)
  smem = pl.BlockSpec(memory_space=pltpu.SMEM)
  out = pl.pallas_call(
      _dense_body,
      out_shape=jax.ShapeDtypeStruct((N, D), jnp.float32),
      in_specs=[vmem, vmem, vmem, vmem, vmem, vmem, smem, vmem, vmem],
      out_specs=vmem,
  )(yp, x, W1, b1, W2, b2, eps, gamma, beta)
  return out


# speedup vs baseline: 5.5649x; 5.5649x over previous
import functools

import jax
import jax.numpy as jnp
from jax import lax
from jax.experimental import pallas as pl
from jax.experimental.pallas import tpu as pltpu
from jax.experimental.pallas import tpu_sc as plsc

NC = 2
NS = 16
NW = NC * NS


def _make_segsum(N, E, D):
  ep = E // NW
  ch = 80
  nchunk = ep // ch
  rch = 80
  nrc = N // rch
  nrc_per_tile = -(-nrc // NS)

  mesh = plsc.VectorSubcoreMesh(core_axis_name="c", subcore_axis_name="s")

  @functools.partial(
      pl.kernel,
      out_type=jax.ShapeDtypeStruct((NC, N, D), jnp.float32),
      mesh=mesh,
      scratch_types=[
          pltpu.VMEM((ch,), jnp.int32),
          pltpu.VMEM((ch,), jnp.int32),
          pltpu.VMEM((ch, D), jnp.float32),
          pltpu.VMEM((rch, D), jnp.float32),
          pltpu.VMEM_SHARED((N, D), jnp.float32),
          pltpu.SemaphoreType.DMA,
      ],
  )
  def segsum(src_hbm, dst_hbm, x_hbm, out_hbm, sidx, didx, rows, zbuf, yacc,
             sem):
    c = lax.axis_index("c")
    s = lax.axis_index("s")
    wid = c * NS + s

    zv = jnp.zeros((16,), jnp.float32)

    def zrow(r, carry):
      for k in range(D // 16):
        zbuf[r, pl.ds(k * 16, 16)] = zv
      return carry

    lax.fori_loop(0, rch, zrow, 0)

    for z in range(nrc_per_tile):
      ci = s + NS * z

      @pl.when(ci < nrc)
      def _():
        pltpu.sync_copy(zbuf, yacc.at[pl.ds(pl.multiple_of(ci * rch, 8), rch)])

    plsc.subcore_barrier()

    ebase = wid * ep

    def edge_chunk(j, carry):
      off = pl.multiple_of(ebase + j * ch, 8)
      pltpu.sync_copy(src_hbm.at[pl.ds(off, ch)], sidx)
      pltpu.sync_copy(dst_hbm.at[pl.ds(off, ch)], didx)
      pltpu.async_copy(x_hbm.at[sidx], rows, sem).wait()
      pltpu.sync_copy(rows, yacc.at[didx], add=True)
      return carry

    lax.fori_loop(0, nchunk, edge_chunk, 0)

    plsc.subcore_barrier()

    for z in range(nrc_per_tile):
      ci = s + NS * z

      @pl.when(ci < nrc)
      def _():
        r0 = pl.multiple_of(ci * rch, 8)
        pltpu.sync_copy(yacc.at[pl.ds(r0, rch)], out_hbm.at[c, pl.ds(r0, rch)])

  return segsum


def _dense_body(yp_ref, x_ref, w1_ref, b1_ref, w2_ref, b2_ref, eps_ref,
                gamma_ref, beta_ref, o_ref):
  n = x_ref.shape[0]
  y = yp_ref[0] + yp_ref[1]
  h = y + (1.0 + eps_ref[0]) * x_ref[...]
  h = lax.dot_general(h, w1_ref[...], (((1,), (1,)), ((), ())),
                      preferred_element_type=jnp.float32)
  h = jnp.maximum(h + b1_ref[...][None, :], 0.0)
  h = lax.dot_general(h, w2_ref[...], (((1,), (1,)), ((), ())),
                      preferred_element_type=jnp.float32)
  h = h + b2_ref[...][None, :]
  mean = jnp.sum(h, axis=0, keepdims=True) * (1.0 / n)
  d = h - mean
  var = jnp.sum(d * d, axis=0, keepdims=True) * (1.0 / n)
  o_ref[...] = d * lax.rsqrt(var + 1e-5) * gamma_ref[...][None, :] \
      + beta_ref[...][None, :]


def kernel(x, edge_index, W1, b1, W2, b2, eps, gamma, beta):
  N, D = x.shape
  E = edge_index.shape[1]
  src = edge_index[0]
  dst = edge_index[1]

  yp = _make_segsum(N, E, D)(src, dst, x)

  vmem = pl.BlockSpec(memory_space=pltpu.VMEM)
  smem = pl.BlockSpec(memory_space=pltpu.SMEM)
  out = pl.pallas_call(
      _dense_body,
      out_shape=jax.ShapeDtypeStruct((N, D), jnp.float32),
      in_specs=[vmem, vmem, vmem, vmem, vmem, vmem, smem, vmem, vmem],
      out_specs=vmem,
  )(yp, x, W1, b1, W2, b2, eps, gamma, beta)
  return out


# speedup vs baseline: 9.7798x; 1.7574x over previous
import functools

import jax
import jax.numpy as jnp
from jax import lax
from jax.experimental import pallas as pl
from jax.experimental.pallas import tpu as pltpu
from jax.experimental.pallas import tpu_sc as plsc

NC = 2
NS = 16
NW = NC * NS


def _make_segsum(N, E, D):
  ep = E // NW
  ch = 40
  nbuf = 5
  npass = ep // (ch * nbuf)
  rch = 40
  nrc = N // rch
  nrc_per_tile = -(-nrc // NS)

  mesh = plsc.VectorSubcoreMesh(core_axis_name="c", subcore_axis_name="s")

  @functools.partial(
      pl.kernel,
      out_type=jax.ShapeDtypeStruct((NC, N, D), jnp.float32),
      mesh=mesh,
      scratch_types=[
          [pltpu.VMEM((nbuf, ch), jnp.int32)] * 2,
          [pltpu.VMEM((nbuf, ch), jnp.int32)] * 2,
          [pltpu.VMEM((ch, D), jnp.float32)] * nbuf,
          pltpu.VMEM_SHARED((N, D), jnp.float32),
          [pltpu.SemaphoreType.DMA] * 2,
          [pltpu.SemaphoreType.DMA] * nbuf,
          [pltpu.SemaphoreType.DMA] * nbuf,
      ],
  )
  def segsum(src_hbm, dst_hbm, x_hbm, out_hbm, sidxb, didxb, rows, yacc,
             isem, gsem, ssem):
    c = lax.axis_index("c")
    s = lax.axis_index("s")
    wid = c * NS + s

    pltpu.async_copy(src_hbm.at[wid, 0], sidxb[0], isem[0])
    pltpu.async_copy(dst_hbm.at[wid, 0], didxb[0], isem[0])

    zv = jnp.zeros((16,), jnp.float32)

    def zrow(r, carry):
      for k in range(D // 16):
        rows[0][r, pl.ds(k * 16, 16)] = zv
      return carry

    lax.fori_loop(0, rch, zrow, 0)

    for z in range(nrc_per_tile):
      ci = s + NS * z

      @pl.when(ci < nrc)
      def _():
        pltpu.sync_copy(rows[0],
                        yacc.at[pl.ds(pl.multiple_of(ci * rch, 8), rch)])

    plsc.subcore_barrier()

    def two_passes(u, carry):
      for q in range(2):
        t = 2 * u + q
        nxt = 1 - q

        @pl.when(t + 1 < npass)
        def _():
          pltpu.async_copy(src_hbm.at[wid, t + 1], sidxb[nxt], isem[nxt])
          pltpu.async_copy(dst_hbm.at[wid, t + 1], didxb[nxt], isem[nxt])

        pltpu.make_async_copy(src_hbm.at[wid, t], sidxb[q], isem[q]).wait()
        pltpu.make_async_copy(dst_hbm.at[wid, t], didxb[q], isem[q]).wait()

        gh = [pltpu.async_copy(x_hbm.at[sidxb[q].at[b]], rows[b], gsem[b])
              for b in range(nbuf)]
        sh = []
        for b in range(nbuf):
          gh[b].wait()
          sh.append(pltpu.async_copy(rows[b], yacc.at[didxb[q].at[b]],
                                     ssem[b], add=True))
        for b in range(nbuf):
          sh[b].wait()
      return carry

    lax.fori_loop(0, npass // 2, two_passes, 0)

    plsc.subcore_barrier()

    for z in range(nrc_per_tile):
      ci = s + NS * z

      @pl.when(ci < nrc)
      def _():
        r0 = pl.multiple_of(ci * rch, 8)
        pltpu.sync_copy(yacc.at[pl.ds(r0, rch)], out_hbm.at[c, pl.ds(r0, rch)])

  return segsum


def _dense_body(yp_ref, x_ref, w1_ref, b1_ref, w2_ref, b2_ref, eps_ref,
                gamma_ref, beta_ref, o_ref):
  n = x_ref.shape[0]
  y = yp_ref[0] + yp_ref[1]
  h = y + (1.0 + eps_ref[0]) * x_ref[...]
  h = lax.dot_general(h, w1_ref[...], (((1,), (1,)), ((), ())),
                      preferred_element_type=jnp.float32)
  h = jnp.maximum(h + b1_ref[...][None, :], 0.0)
  h = lax.dot_general(h, w2_ref[...], (((1,), (1,)), ((), ())),
                      preferred_element_type=jnp.float32)
  h = h + b2_ref[...][None, :]
  mean = jnp.sum(h, axis=0, keepdims=True) * (1.0 / n)
  d = h - mean
  var = jnp.sum(d * d, axis=0, keepdims=True) * (1.0 / n)
  o_ref[...] = d * lax.rsqrt(var + 1e-5) * gamma_ref[...][None, :] \
      + beta_ref[...][None, :]


def kernel(x, edge_index, W1, b1, W2, b2, eps, gamma, beta):
  N, D = x.shape
  E = edge_index.shape[1]
  ep = E // NW
  ch, nbuf = 40, 5
  src = edge_index[0].reshape(NW, ep // (ch * nbuf), nbuf, ch)
  dst = edge_index[1].reshape(NW, ep // (ch * nbuf), nbuf, ch)

  yp = _make_segsum(N, E, D)(src, dst, x)

  vmem = pl.BlockSpec(memory_space=pltpu.VMEM)
  smem = pl.BlockSpec(memory_space=pltpu.SMEM)
  out = pl.pallas_call(
      _dense_body,
      out_shape=jax.ShapeDtypeStruct((N, D), jnp.float32),
      in_specs=[vmem, vmem, vmem, vmem, vmem, vmem, smem, vmem, vmem],
      out_specs=vmem,
  )(yp, x, W1, b1, W2, b2, eps, gamma, beta)
  return out
